# baseline (speedup 1.0000x reference)
import functools

import jax
import jax.numpy as jnp
from jax import lax
from jax.experimental import pallas as pl
from jax.experimental.pallas import tpu as pltpu
from jax.experimental.pallas import tpu_sc as plsc

D = 64
K = 10
L = 16
NC = 2
NS = 16
NW = NC * NS
CB = 64
W = 2 * D


def _sc_scores(tidx_h, tidx_o, cidx_h, cidx_o, nidx_h, nidx_o,
               target_pairs, context_pairs):
    B = tidx_h.shape[0]
    bpw = B // NW
    nchunks = bpw // CB

    mesh = plsc.VectorSubcoreMesh(
        core_axis_name="c", subcore_axis_name="s", num_cores=NC,
        num_subcores=NS)

    @functools.partial(
        pl.kernel,
        out_type=(
            jax.ShapeDtypeStruct((B,), jnp.float32),
            jax.ShapeDtypeStruct((NW, K * bpw), jnp.float32),
        ),
        mesh=mesh,
        scratch_types=[
            pltpu.VMEM((bpw,), jnp.int32),
            pltpu.VMEM((bpw + L,), jnp.int32),
            pltpu.VMEM((bpw,), jnp.int32),
            pltpu.VMEM((bpw + L,), jnp.int32),
            pltpu.VMEM((K, bpw), jnp.int32),
            pltpu.VMEM((K, bpw + L), jnp.int32),
            pltpu.VMEM((CB, W), jnp.float32),
            pltpu.VMEM((CB, W), jnp.float32),
            pltpu.VMEM((K, CB, W), jnp.float32),
            pltpu.VMEM((bpw + L,), jnp.float32),
            pltpu.VMEM((K * bpw + L,), jnp.float32),
            pltpu.SemaphoreType.DMA,
        ],
        compiler_params=pltpu.CompilerParams(needs_layout_passes=False),
    )
    def sc_kernel(tih_hbm, tio_hbm, cih_hbm, cio_hbm, nih_hbm, nio_hbm,
                  temb_hbm, cemb_hbm, pos_hbm, neg_hbm,
                  tih, tio, cih, cio, nih, nio,
                  trows, crows, nrows, posv, negv, sem):
        wid = lax.axis_index("s") * NC + lax.axis_index("c")
        base = wid * bpw
        last_lane = lax.iota(jnp.int32, L) == (L - 1)

        pltpu.sync_copy(tih_hbm.at[pl.ds(base, bpw)], tih)
        pltpu.sync_copy(tio_hbm.at[pl.ds(base, bpw)], tio.at[pl.ds(0, bpw)])
        pltpu.sync_copy(cih_hbm.at[pl.ds(base, bpw)], cih)
        pltpu.sync_copy(cio_hbm.at[pl.ds(base, bpw)], cio.at[pl.ds(0, bpw)])
        pltpu.sync_copy(nih_hbm.at[:, pl.ds(base, bpw)], nih)
        pltpu.sync_copy(nio_hbm.at[:, pl.ds(base, bpw)],
                        nio.at[:, pl.ds(0, bpw)])

        for c in range(nchunks):
            cb0 = c * CB
            copies = [
                pltpu.async_copy(temb_hbm.at[tih.at[pl.ds(cb0, CB)]], trows,
                                 sem),
                pltpu.async_copy(cemb_hbm.at[cih.at[pl.ds(cb0, CB)]], crows,
                                 sem),
            ]
            for k in range(K):
                copies.append(
                    pltpu.async_copy(cemb_hbm.at[nih.at[k, pl.ds(cb0, CB)]],
                                     nrows.at[k], sem))
            for cp in copies:
                cp.wait()

            def body(i, carry):
                po = tio[pl.ds(cb0 + i, L)][0]
                co = cio[pl.ds(cb0 + i, L)][0]
                t = [trows[i, pl.ds(po + j * L, L)] for j in range(D // L)]
                cv = [crows[i, pl.ds(co + j * L, L)] for j in range(D // L)]
                p = t[0] * cv[0] + t[1] * cv[1] + t[2] * cv[2] + t[3] * cv[3]
                plsc.store_compressed(posv.at[pl.ds(cb0 + i, L)],
                                      plsc.cumsum(p), mask=last_lane)
                for k in range(K):
                    no = nio[k, pl.ds(cb0 + i, L)][0]
                    n = [nrows[k, i, pl.ds(no + j * L, L)]
                         for j in range(D // L)]
                    q = n[0] * t[0] + n[1] * t[1] + n[2] * t[2] + n[3] * t[3]
                    plsc.store_compressed(
                        negv.at[pl.ds(k * bpw + cb0 + i, L)],
                        plsc.cumsum(q), mask=last_lane)
                return carry

            lax.fori_loop(0, CB, body, 0)

        pltpu.sync_copy(posv.at[pl.ds(0, bpw)], pos_hbm.at[pl.ds(base, bpw)])
        pltpu.sync_copy(negv.at[pl.ds(0, K * bpw)], neg_hbm.at[wid])

    return sc_kernel(tidx_h, tidx_o, cidx_h, cidx_o, nidx_h, nidx_o,
                     target_pairs, context_pairs)


def _loss_tc(pos_scores, neg_scores):
    pos2 = pos_scores.reshape(-1, 128)
    neg2 = neg_scores.reshape(-1, 128)

    def body(pos_ref, neg_ref, out_ref):
        p = pos_ref[...]
        n = neg_ref[...]
        ls_p = jnp.minimum(p, 0.0) - jnp.log1p(jnp.exp(-jnp.abs(p)))
        ls_n = jnp.minimum(-n, 0.0) - jnp.log1p(jnp.exp(-jnp.abs(n)))
        out_ref[0, 0] = -(jnp.mean(ls_p) + jnp.mean(ls_n))

    out = pl.pallas_call(
        body,
        out_shape=jax.ShapeDtypeStruct((1, 1), jnp.float32),
        out_specs=pl.BlockSpec(memory_space=pltpu.SMEM),
    )(pos2, neg2)
    return out[0, 0]


def kernel(target_idx, context_idx, negative_idx, target_embeddings,
           context_embeddings):
    V = target_embeddings.shape[0]
    target_pairs = target_embeddings.reshape(V // 2, W)
    context_pairs = context_embeddings.reshape(V // 2, W)
    tidx_h = target_idx >> 1
    tidx_o = (target_idx & 1) * D
    cidx_h = context_idx >> 1
    cidx_o = (context_idx & 1) * D
    nidx_h = (negative_idx >> 1).T
    nidx_o = ((negative_idx & 1) * D).T
    pos_scores, neg_scores = _sc_scores(
        tidx_h, tidx_o, cidx_h, cidx_o, nidx_h, nidx_o,
        target_pairs, context_pairs)
    return _loss_tc(pos_scores, neg_scores)
